# baseline (speedup 1.0000x reference)
import functools

import jax
import jax.numpy as jnp
from jax import lax
from jax.experimental import pallas as pl
from jax.experimental.pallas import tpu as pltpu
from jax.experimental.pallas import tpu_sc as plsc

B = 8192
D = 256
N = 8192
BM = 1024
MT = B // BM

SC_CORES = 2
SC_SUBCORES = 16
SC_WORKERS = SC_CORES * SC_SUBCORES
ROWS_PER_WORKER = B // SC_WORKERS
IDX_CHUNK = 128


NC = 4
CH = N // NC


def _vq_distance_body(z_ref, e_ref, idx_ref, loss_ref, e2_ref, et_ref):
    m = pl.program_id(0)
    z = z_ref[...]

    @pl.when(m == 0)
    def _():
        e = e_ref[...]
        et_ref[...] = e.T
        e2_ref[...] = jnp.sum(e * e, axis=1).reshape(1, N)

    zz = jnp.sum(z * z, axis=1, keepdims=True)
    it = lax.broadcasted_iota(jnp.int32, (1, CH), 1).astype(jnp.float32)
    z2 = z + z

    def chunk_dot(c):
        return lax.dot_general(
            z2, et_ref[:, pl.ds(c * CH, CH)], (((1,), (0,)), ((), ())),
            preferred_element_type=jnp.float32,
            precision=lax.Precision.DEFAULT)

    run_min = run_arg = None
    prods = [chunk_dot(0)] + [None] * (NC - 1)
    for c in range(NC):
        if c + 1 < NC:
            prods[c + 1] = chunk_dot(c + 1)
        e2c = e2_ref[:, pl.ds(c * CH, CH)]
        dc = zz + e2c - prods[c]
        tm = jnp.min(dc, axis=1, keepdims=True)
        tg = jnp.min(jnp.where(dc == tm, it, jnp.float32(CH)),
                     axis=1, keepdims=True) + jnp.float32(c * CH)
        if c == 0:
            run_min, run_arg = tm, tg
        else:
            better = tm < run_min
            run_arg = jnp.where(better, tg, run_arg)
            run_min = jnp.where(better, tm, run_min)

    idx_ref[...] = run_arg.astype(jnp.int32)
    part = jnp.sum(run_min)

    @pl.when(m == 0)
    def _():
        loss_ref[...] = jnp.zeros((1, 1), jnp.float32) + part

    @pl.when(m > 0)
    def _():
        loss_ref[...] = loss_ref[...] + part

    @pl.when(m == MT - 1)
    def _():
        loss_ref[...] = loss_ref[...] * jnp.float32(1.25 / (B * D))


_distance_call = pl.pallas_call(
    _vq_distance_body,
    grid=(MT,),
    in_specs=[
        pl.BlockSpec((BM, D), lambda m: (m, 0)),
        pl.BlockSpec((N, D), lambda m: (0, 0)),
    ],
    out_specs=[
        pl.BlockSpec((BM, 1), lambda m: (m, 0)),
        pl.BlockSpec((1, 1), lambda m: (0, 0)),
    ],
    out_shape=[
        jax.ShapeDtypeStruct((B, 1), jnp.int32),
        jax.ShapeDtypeStruct((1, 1), jnp.float32),
    ],
    scratch_shapes=[
        pltpu.VMEM((1, N), jnp.float32),
        pltpu.VMEM((D, N), jnp.float32),
    ],
    compiler_params=pltpu.CompilerParams(
        dimension_semantics=("arbitrary",)),
)


def _gather_body(table_hbm, idx_hbm, out_hbm, idx_v, rows_v, sem, out_sem):
    wid = lax.axis_index("s") * SC_CORES + lax.axis_index("c")
    base = wid * ROWS_PER_WORKER
    pltpu.sync_copy(idx_hbm.at[pl.ds(base, ROWS_PER_WORKER)], idx_v)
    nch = ROWS_PER_WORKER // IDX_CHUNK
    gathers = [pltpu.async_copy(
        table_hbm.at[idx_v.at[pl.ds(j * IDX_CHUNK, IDX_CHUNK)]],
        rows_v.at[pl.ds(j * IDX_CHUNK, IDX_CHUNK)],
        sem) for j in range(nch)]
    outs = []
    for j in range(nch):
        gathers[j].wait()
        outs.append(pltpu.async_copy(
            rows_v.at[pl.ds(j * IDX_CHUNK, IDX_CHUNK)],
            out_hbm.at[pl.ds(base + j * IDX_CHUNK, IDX_CHUNK)],
            out_sem))
    for cp in outs:
        cp.wait()


@functools.cache
def _sc_gather():
    return pl.kernel(
        _gather_body,
        out_type=jax.ShapeDtypeStruct((B, D), jnp.float32),
        mesh=plsc.VectorSubcoreMesh(
            core_axis_name="c", subcore_axis_name="s"),
        scratch_types=[
            pltpu.VMEM((ROWS_PER_WORKER,), jnp.int32),
            pltpu.VMEM((ROWS_PER_WORKER, D), jnp.float32),
            pltpu.SemaphoreType.DMA,
            pltpu.SemaphoreType.DMA,
        ],
    )


def kernel(inputs, embedding_weight):
    idx2d, loss_sum = _distance_call(inputs, embedding_weight)
    indices = idx2d.reshape(B)
    quantized = _sc_gather()(embedding_weight, indices)
    return quantized, loss_sum.reshape(()), indices


# speedup vs baseline: 1.0049x; 1.0049x over previous
import functools

import jax
import jax.numpy as jnp
from jax import lax
from jax.experimental import pallas as pl
from jax.experimental.pallas import tpu as pltpu
from jax.experimental.pallas import tpu_sc as plsc

B = 8192
D = 256
N = 8192
BM = 1024
MT = B // BM

SC_CORES = 2
SC_SUBCORES = 16
SC_WORKERS = SC_CORES * SC_SUBCORES
ROWS_PER_WORKER = B // SC_WORKERS
IDX_CHUNK = 128


NC = 4
CH = N // NC


def _vq_distance_body(z_ref, e_ref, idx_ref, loss_ref, e2_ref, et_ref):
    m = pl.program_id(0)
    z = z_ref[...]

    @pl.when(m == 0)
    def _():
        e = e_ref[...]
        et_ref[...] = (e + e).T
        e2_ref[...] = jnp.sum(e * e, axis=1).reshape(1, N)

    zz = jnp.sum(z * z, axis=1, keepdims=True)
    it = lax.broadcasted_iota(jnp.int32, (1, CH), 1).astype(jnp.float32)

    def chunk_dot(c):
        return lax.dot_general(
            z, et_ref[:, pl.ds(c * CH, CH)], (((1,), (0,)), ((), ())),
            preferred_element_type=jnp.float32,
            precision=lax.Precision.DEFAULT)

    run_min = run_arg = None
    prods = [chunk_dot(0)] + [None] * (NC - 1)
    for c in range(NC):
        if c + 1 < NC:
            prods[c + 1] = chunk_dot(c + 1)
        e2c = e2_ref[:, pl.ds(c * CH, CH)]
        dc = zz + e2c - prods[c]
        tm = jnp.min(dc, axis=1, keepdims=True)
        tg = jnp.min(jnp.where(dc == tm, it, jnp.float32(CH)),
                     axis=1, keepdims=True) + jnp.float32(c * CH)
        if c == 0:
            run_min, run_arg = tm, tg
        else:
            better = tm < run_min
            run_arg = jnp.where(better, tg, run_arg)
            run_min = jnp.where(better, tm, run_min)

    idx_ref[...] = run_arg.astype(jnp.int32)
    part = jnp.sum(run_min)

    @pl.when(m == 0)
    def _():
        loss_ref[...] = jnp.zeros((1, 1), jnp.float32) + part

    @pl.when(m > 0)
    def _():
        loss_ref[...] = loss_ref[...] + part

    @pl.when(m == MT - 1)
    def _():
        loss_ref[...] = loss_ref[...] * jnp.float32(1.25 / (B * D))


_distance_call = pl.pallas_call(
    _vq_distance_body,
    grid=(MT,),
    in_specs=[
        pl.BlockSpec((BM, D), lambda m: (m, 0)),
        pl.BlockSpec((N, D), lambda m: (0, 0)),
    ],
    out_specs=[
        pl.BlockSpec((BM, 1), lambda m: (m, 0)),
        pl.BlockSpec((1, 1), lambda m: (0, 0)),
    ],
    out_shape=[
        jax.ShapeDtypeStruct((B, 1), jnp.int32),
        jax.ShapeDtypeStruct((1, 1), jnp.float32),
    ],
    scratch_shapes=[
        pltpu.VMEM((1, N), jnp.float32),
        pltpu.VMEM((D, N), jnp.float32),
    ],
    compiler_params=pltpu.CompilerParams(
        dimension_semantics=("arbitrary",)),
)


def _gather_body(table_hbm, idx_hbm, out_hbm, idx_v, rows_v, sem, out_sem):
    wid = lax.axis_index("s") * SC_CORES + lax.axis_index("c")
    base = wid * ROWS_PER_WORKER
    pltpu.sync_copy(idx_hbm.at[pl.ds(base, ROWS_PER_WORKER)], idx_v)
    nch = ROWS_PER_WORKER // IDX_CHUNK
    gathers = [pltpu.async_copy(
        table_hbm.at[idx_v.at[pl.ds(j * IDX_CHUNK, IDX_CHUNK)]],
        rows_v.at[pl.ds(j * IDX_CHUNK, IDX_CHUNK)],
        sem) for j in range(nch)]
    outs = []
    for j in range(nch):
        gathers[j].wait()
        outs.append(pltpu.async_copy(
            rows_v.at[pl.ds(j * IDX_CHUNK, IDX_CHUNK)],
            out_hbm.at[pl.ds(base + j * IDX_CHUNK, IDX_CHUNK)],
            out_sem))
    for cp in outs:
        cp.wait()


@functools.cache
def _sc_gather():
    return pl.kernel(
        _gather_body,
        out_type=jax.ShapeDtypeStruct((B, D), jnp.float32),
        mesh=plsc.VectorSubcoreMesh(
            core_axis_name="c", subcore_axis_name="s"),
        scratch_types=[
            pltpu.VMEM((ROWS_PER_WORKER,), jnp.int32),
            pltpu.VMEM((ROWS_PER_WORKER, D), jnp.float32),
            pltpu.SemaphoreType.DMA,
            pltpu.SemaphoreType.DMA,
        ],
    )


def kernel(inputs, embedding_weight):
    idx2d, loss_sum = _distance_call(inputs, embedding_weight)
    indices = idx2d.reshape(B)
    quantized = _sc_gather()(embedding_weight, indices)
    return quantized, loss_sum.reshape(()), indices


# speedup vs baseline: 1.0742x; 1.0689x over previous
import functools

import jax
import jax.numpy as jnp
from jax import lax
from jax.experimental import pallas as pl
from jax.experimental.pallas import tpu as pltpu
from jax.experimental.pallas import tpu_sc as plsc

B = 8192
D = 256
N = 8192
BM = 1024
MT = B // BM

SC_CORES = 2
SC_SUBCORES = 16
SC_WORKERS = SC_CORES * SC_SUBCORES
ROWS_PER_WORKER = B // SC_WORKERS
IDX_CHUNK = 128


NC = 4
CH = N // NC


def _vq_distance_body(z_ref, e_ref, idx_ref, loss_ref, e2_ref, et_ref):
    m = pl.program_id(0)
    z = z_ref[...]

    @pl.when(m == 0)
    def _():
        e = e_ref[...]
        et_ref[...] = e.T
        e2_ref[...] = jnp.sum(e * e, axis=1).reshape(1, N)

    zz = jnp.sum(z * z, axis=1, keepdims=True)
    it = lax.broadcasted_iota(jnp.int32, (1, CH), 1).astype(jnp.float32)

    def chunk_dot(c):
        return lax.dot_general(
            z, et_ref[:, pl.ds(c * CH, CH)], (((1,), (0,)), ((), ())),
            preferred_element_type=jnp.float32,
            precision=lax.Precision.DEFAULT)

    run_min = run_arg = None
    prods = [chunk_dot(0)] + [None] * (NC - 1)
    for c in range(NC):
        if c + 1 < NC:
            prods[c + 1] = chunk_dot(c + 1)
        e2c = e2_ref[:, pl.ds(c * CH, CH)]
        dc = zz + e2c - 2.0 * prods[c]
        tm = jnp.min(dc, axis=1, keepdims=True)
        tg = jnp.min(jnp.where(dc == tm, it, jnp.float32(CH)),
                     axis=1, keepdims=True) + jnp.float32(c * CH)
        if c == 0:
            run_min, run_arg = tm, tg
        else:
            better = tm < run_min
            run_arg = jnp.where(better, tg, run_arg)
            run_min = jnp.where(better, tm, run_min)

    idx_ref[...] = run_arg.astype(jnp.int32)
    part = jnp.sum(run_min)

    @pl.when(m == 0)
    def _():
        loss_ref[...] = jnp.zeros((1, 1), jnp.float32) + part

    @pl.when(m > 0)
    def _():
        loss_ref[...] = loss_ref[...] + part

    @pl.when(m == MT - 1)
    def _():
        loss_ref[...] = loss_ref[...] * jnp.float32(1.25 / (B * D))


_distance_call = pl.pallas_call(
    _vq_distance_body,
    grid=(MT,),
    in_specs=[
        pl.BlockSpec((BM, D), lambda m: (m, 0)),
        pl.BlockSpec((N, D), lambda m: (0, 0)),
    ],
    out_specs=[
        pl.BlockSpec((BM, 1), lambda m: (m, 0)),
        pl.BlockSpec((1, 1), lambda m: (0, 0)),
    ],
    out_shape=[
        jax.ShapeDtypeStruct((B, 1), jnp.int32),
        jax.ShapeDtypeStruct((1, 1), jnp.float32),
    ],
    scratch_shapes=[
        pltpu.VMEM((1, N), jnp.float32),
        pltpu.VMEM((D, N), jnp.float32),
    ],
    compiler_params=pltpu.CompilerParams(
        dimension_semantics=("arbitrary",)),
)


def _gather_body(table_hbm, idx_hbm, out_hbm, idx_v, rows_v, sem):
    wid = lax.axis_index("s") * SC_CORES + lax.axis_index("c")
    base = wid * ROWS_PER_WORKER
    pltpu.sync_copy(idx_hbm.at[pl.ds(base, ROWS_PER_WORKER)], idx_v)
    nch = ROWS_PER_WORKER // IDX_CHUNK
    gathers = [pltpu.async_copy(
        table_hbm.at[idx_v.at[pl.ds(j * IDX_CHUNK, IDX_CHUNK)]],
        rows_v.at[pl.ds(j * IDX_CHUNK, IDX_CHUNK)],
        sem) for j in range(nch)]
    for cp in gathers:
        cp.wait()
    pltpu.sync_copy(rows_v, out_hbm.at[pl.ds(base, ROWS_PER_WORKER)])


@functools.cache
def _sc_gather():
    return pl.kernel(
        _gather_body,
        out_type=jax.ShapeDtypeStruct((B, D), jnp.float32),
        mesh=plsc.VectorSubcoreMesh(
            core_axis_name="c", subcore_axis_name="s"),
        scratch_types=[
            pltpu.VMEM((ROWS_PER_WORKER,), jnp.int32),
            pltpu.VMEM((ROWS_PER_WORKER, D), jnp.float32),
            pltpu.SemaphoreType.DMA,
        ],
    )


def kernel(inputs, embedding_weight):
    idx2d, loss_sum = _distance_call(inputs, embedding_weight)
    indices = idx2d.reshape(B)
    quantized = _sc_gather()(embedding_weight, indices)
    return quantized, loss_sum.reshape(()), indices
